# baseline (speedup 1.0000x reference)
import functools

import jax
import jax.numpy as jnp
from jax import lax
from jax.experimental import pallas as pl
from jax.experimental.pallas import tpu as pltpu
from jax.experimental.pallas import tpu_sc as plsc

_N = 10000
_NP = 10240
_H = 128
_OUT = 32
_R = 4
_NB = 2
_E = 160000
_NCORES = 2
_NSUB = 16
_NW = _NCORES * _NSUB
_EPW = 5120
_ECH = 128
_NCH = _EPW // _ECH
_RPT = _NP // _NSUB
_IPW = _NP // _NW
_ICH = 80
_INCH = _IPW // _ICH

_mesh = plsc.VectorSubcoreMesh(core_axis_name="c", subcore_axis_name="s")


_K = 2
_NG = _NCH // _K


def _make_gsa(F):
    @functools.partial(
        pl.kernel, mesh=_mesh,
        out_type=jax.ShapeDtypeStruct((_NCORES, _NP, F), jnp.float32),
        scratch_types=[
            pltpu.VMEM((_NCH, _ECH), jnp.int32),
            pltpu.VMEM((_NCH, _ECH), jnp.int32),
            pltpu.VMEM_SHARED((_NP, F), jnp.float32),
        ]
        + [pltpu.VMEM((_ECH, F), jnp.float32) for _ in range(_K)]
        + [pltpu.SemaphoreType.DMA for _ in range(2 * _K)],
    )
    def gsa(hw_hbm, gidx_hbm, didx_hbm, zeros_hbm, out_hbm,
            gidx_v, didx_v, acc_shared, *bufs_and_sems):
        rows = bufs_and_sems[:_K]
        gsem = bufs_and_sems[_K:2 * _K]
        ssem = bufs_and_sems[2 * _K:3 * _K]
        c = lax.axis_index("c")
        s = lax.axis_index("s")
        w = c * _NSUB + s
        pltpu.sync_copy(zeros_hbm.at[pl.ds(s * _RPT, _RPT)],
                        acc_shared.at[pl.ds(s * _RPT, _RPT)])
        pltpu.sync_copy(gidx_hbm.at[w], gidx_v)
        pltpu.sync_copy(didx_hbm.at[w], didx_v)
        plsc.subcore_barrier()

        for b in range(_K):
            pltpu.async_copy(hw_hbm.at[gidx_v.at[b]], rows[b], gsem[b])

        def group(i, carry):
            for b in range(_K):
                pltpu.make_async_copy(
                    hw_hbm.at[gidx_v.at[0]], rows[b], gsem[b]).wait()
                pltpu.async_copy(rows[b], acc_shared.at[didx_v.at[i * _K + b]],
                                 ssem[b], add=True)
            for b in range(_K):
                pltpu.make_async_copy(
                    rows[b], acc_shared.at[didx_v.at[0]], ssem[b]).wait()

                @pl.when(i + 1 < _NG)
                def _():
                    pltpu.async_copy(
                        hw_hbm.at[gidx_v.at[(i + 1) * _K + b]],
                        rows[b], gsem[b])
            return carry

        lax.fori_loop(0, _NG, group, 0)
        plsc.subcore_barrier()
        pltpu.sync_copy(acc_shared.at[pl.ds(s * _RPT, _RPT)],
                        out_hbm.at[c, pl.ds(s * _RPT, _RPT)])

    return gsa


_gsa128 = _make_gsa(_H)

_BN = 1024
_NBLK = _NP // _BN


def _tg_body(fp_ref, w_ref, g_ref):
    fp = fp_ref[...]
    for r in range(_R):
        g_ref[r] = jnp.dot(fp, w_ref[r], preferred_element_type=jnp.float32)


def _tg(feat_in, w0):
    fp = jnp.pad(feat_in, ((0, 128 - feat_in.shape[0]), (0, 0)))
    return pl.pallas_call(
        _tg_body,
        in_specs=[
            pl.BlockSpec((128, _H), lambda: (0, 0)),
            pl.BlockSpec((_R, _H, _H), lambda: (0, 0, 0)),
        ],
        out_specs=pl.BlockSpec((_R, 128, _H), lambda: (0, 0, 0)),
        out_shape=jax.ShapeDtypeStruct((_R, 128, _H), jnp.float32),
    )(fp, w0)


def _tmid_body(p_ref, w_ref, hw_ref):
    h = jnp.maximum(p_ref[0] + p_ref[1], 0.0)
    for r in range(_R):
        hw_ref[r] = jnp.dot(h, w_ref[r], preferred_element_type=jnp.float32)


def _tmid(p, w):
    fo = w.shape[2]
    return pl.pallas_call(
        _tmid_body,
        grid=(_NBLK,),
        in_specs=[
            pl.BlockSpec((_NCORES, _BN, _H), lambda i: (0, i, 0)),
            pl.BlockSpec((_R, _H, fo), lambda i: (0, 0, 0)),
        ],
        out_specs=pl.BlockSpec((_R, _BN, fo), lambda i: (0, i, 0)),
        out_shape=jax.ShapeDtypeStruct((_R, _NP, fo), jnp.float32),
    )(p, w)


def _softmax_body(p_ref, o_ref):
    x = p_ref[0][:, :_OUT] + p_ref[1][:, :_OUT]
    m = jnp.max(x, axis=1, keepdims=True)
    e = jnp.exp(x - m)
    o_ref[...] = e / jnp.sum(e, axis=1, keepdims=True)


def _softmax(p):
    return pl.pallas_call(
        _softmax_body,
        grid=(_NBLK,),
        in_specs=[pl.BlockSpec((_NCORES, _BN, _H), lambda i: (0, i, 0))],
        out_specs=pl.BlockSpec((_BN, _OUT), lambda i: (i, 0)),
        out_shape=jax.ShapeDtypeStruct((_NP, _OUT), jnp.float32),
    )(p)


def _basis(C, W, in_f, out_f):
    w = W.reshape(in_f, _NB, out_f)
    w = jnp.matmul(C, w)
    return w.reshape(_R, in_f, out_f)


def kernel(all_class_names, node_states, all_edge_ids, all_edge_types,
           mask_nodes, mask_edges, feat_in, W0, C0, W1, C1, W2, C2):
    ids = all_class_names[0].astype(jnp.int32)
    src = all_edge_ids[0, :, 0].astype(jnp.int32)
    dst = all_edge_ids[0, :, 1].astype(jnp.int32)
    rel = all_edge_types[0].astype(jnp.int32)

    epw0 = _E // _NW
    gidx = (rel * _NP + src).reshape(_NW, epw0)
    gidx = jnp.pad(gidx, ((0, 0), (0, _EPW - epw0))).reshape(_NW, _NCH, _ECH)
    gidx0 = (rel * 128 + ids[src]).reshape(_NW, epw0)
    gidx0 = jnp.pad(gidx0, ((0, 0), (0, _EPW - epw0))).reshape(_NW, _NCH, _ECH)
    didx = jnp.pad(dst.reshape(_NW, epw0), ((0, 0), (0, _EPW - epw0)),
                   constant_values=_N).reshape(_NW, _NCH, _ECH)

    z128 = jnp.zeros((_NP, _H), jnp.float32)
    w0 = _basis(C0, W0, _H, _H)
    w1 = _basis(C1, W1, _H, _H)
    w2 = jnp.pad(_basis(C2, W2, _H, _OUT), ((0, 0), (0, 0), (0, _H - _OUT)))

    g0 = _tg(feat_in, w0).reshape(_R * 128, _H)
    p0 = _gsa128(g0, gidx0, didx, z128)
    hw1 = _tmid(p0, w1).reshape(_R * _NP, _H)
    p1 = _gsa128(hw1, gidx, didx, z128)
    hw2 = _tmid(p1, w2).reshape(_R * _NP, _H)
    p2 = _gsa128(hw2, gidx, didx, z128)
    out = _softmax(p2)
    return out[:_N][None]


# speedup vs baseline: 2.0696x; 2.0696x over previous
import functools

import jax
import jax.numpy as jnp
from jax import lax
from jax.experimental import pallas as pl
from jax.experimental.pallas import tpu as pltpu
from jax.experimental.pallas import tpu_sc as plsc

_N = 10000
_NP = 10240
_H = 128
_OUT = 32
_R = 4
_NB = 2
_E = 160000
_NCORES = 2
_NSUB = 16
_NW = _NCORES * _NSUB
_EPW = 5120
_ECH = 128
_NCH = _EPW // _ECH
_RPT = _NP // _NSUB
_IPW = _NP // _NW
_ICH = 80
_INCH = _IPW // _ICH

_mesh = plsc.VectorSubcoreMesh(core_axis_name="c", subcore_axis_name="s")


@functools.partial(
    pl.kernel, mesh=_mesh,
    out_type=jax.ShapeDtypeStruct((_NP, _H), jnp.float32),
    scratch_types=[
        pltpu.VMEM((_INCH, _ICH), jnp.int32),
        pltpu.VMEM((_ICH, _H), jnp.float32),
        pltpu.SemaphoreType.DMA,
    ],
)
def _emb(table_hbm, idx_hbm, out_hbm, idx_v, rows_v, sem):
    c = lax.axis_index("c")
    s = lax.axis_index("s")
    w = c * _NSUB + s
    pltpu.sync_copy(idx_hbm.at[w], idx_v)
    for j in range(_INCH):
        pltpu.async_copy(table_hbm.at[idx_v.at[j]], rows_v, sem).wait()
        pltpu.sync_copy(rows_v, out_hbm.at[pl.ds(w * _IPW + j * _ICH, _ICH)])


_K = 2
_NG = _NCH // _K


def _make_gsa(F):
    @functools.partial(
        pl.kernel, mesh=_mesh,
        out_type=jax.ShapeDtypeStruct((_NCORES, _NP, F), jnp.float32),
        scratch_types=[
            pltpu.VMEM((_NCH, _ECH), jnp.int32),
            pltpu.VMEM((_NCH, _ECH), jnp.int32),
            pltpu.VMEM_SHARED((_NP, F), jnp.float32),
        ]
        + [pltpu.VMEM((_ECH, F), jnp.float32) for _ in range(_K)]
        + [pltpu.SemaphoreType.DMA for _ in range(2 * _K)],
    )
    def gsa(hw_hbm, gidx_hbm, didx_hbm, zeros_hbm, out_hbm,
            gidx_v, didx_v, acc_shared, *bufs_and_sems):
        rows = bufs_and_sems[:_K]
        gsem = bufs_and_sems[_K:2 * _K]
        ssem = bufs_and_sems[2 * _K:3 * _K]
        c = lax.axis_index("c")
        s = lax.axis_index("s")
        w = c * _NSUB + s
        pltpu.sync_copy(zeros_hbm.at[pl.ds(s * _RPT, _RPT)],
                        acc_shared.at[pl.ds(s * _RPT, _RPT)])
        pltpu.sync_copy(gidx_hbm.at[w], gidx_v)
        pltpu.sync_copy(didx_hbm.at[w], didx_v)
        plsc.subcore_barrier()

        for b in range(_K):
            pltpu.async_copy(hw_hbm.at[gidx_v.at[b]], rows[b], gsem[b])

        def group(i, carry):
            for b in range(_K):
                pltpu.make_async_copy(
                    hw_hbm.at[gidx_v.at[0]], rows[b], gsem[b]).wait()
                pltpu.async_copy(rows[b], acc_shared.at[didx_v.at[i * _K + b]],
                                 ssem[b], add=True)
            for b in range(_K):
                pltpu.make_async_copy(
                    rows[b], acc_shared.at[didx_v.at[0]], ssem[b]).wait()

                @pl.when(i + 1 < _NG)
                def _():
                    pltpu.async_copy(
                        hw_hbm.at[gidx_v.at[(i + 1) * _K + b]],
                        rows[b], gsem[b])
            return carry

        lax.fori_loop(0, _NG, group, 0)
        plsc.subcore_barrier()
        pltpu.sync_copy(acc_shared.at[pl.ds(s * _RPT, _RPT)],
                        out_hbm.at[c, pl.ds(s * _RPT, _RPT)])

    return gsa


_gsa128 = _make_gsa(_H)

_BN = 1024
_NBLK = _NP // _BN


def _tfirst_body(h_ref, w_ref, hw_ref):
    h = h_ref[...]
    for r in range(_R):
        hw_ref[r] = jnp.dot(h, w_ref[r], preferred_element_type=jnp.float32)


def _tfirst(h0, w):
    fo = w.shape[2]
    return pl.pallas_call(
        _tfirst_body,
        grid=(_NBLK,),
        in_specs=[
            pl.BlockSpec((_BN, _H), lambda i: (i, 0)),
            pl.BlockSpec((_R, _H, fo), lambda i: (0, 0, 0)),
        ],
        out_specs=pl.BlockSpec((_R, _BN, fo), lambda i: (0, i, 0)),
        out_shape=jax.ShapeDtypeStruct((_R, _NP, fo), jnp.float32),
    )(h0, w)


def _tmid_body(p_ref, w_ref, hw_ref):
    h = jnp.maximum(p_ref[0] + p_ref[1], 0.0)
    for r in range(_R):
        hw_ref[r] = jnp.dot(h, w_ref[r], preferred_element_type=jnp.float32)


def _tmid(p, w):
    fo = w.shape[2]
    return pl.pallas_call(
        _tmid_body,
        grid=(_NBLK,),
        in_specs=[
            pl.BlockSpec((_NCORES, _BN, _H), lambda i: (0, i, 0)),
            pl.BlockSpec((_R, _H, fo), lambda i: (0, 0, 0)),
        ],
        out_specs=pl.BlockSpec((_R, _BN, fo), lambda i: (0, i, 0)),
        out_shape=jax.ShapeDtypeStruct((_R, _NP, fo), jnp.float32),
    )(p, w)


def _softmax_body(p_ref, o_ref):
    x = p_ref[0][:, :_OUT] + p_ref[1][:, :_OUT]
    m = jnp.max(x, axis=1, keepdims=True)
    e = jnp.exp(x - m)
    o_ref[...] = e / jnp.sum(e, axis=1, keepdims=True)


def _softmax(p):
    return pl.pallas_call(
        _softmax_body,
        grid=(_NBLK,),
        in_specs=[pl.BlockSpec((_NCORES, _BN, _H), lambda i: (0, i, 0))],
        out_specs=pl.BlockSpec((_BN, _OUT), lambda i: (i, 0)),
        out_shape=jax.ShapeDtypeStruct((_NP, _OUT), jnp.float32),
    )(p)


def _basis(C, W, in_f, out_f):
    w = W.reshape(in_f, _NB, out_f)
    w = jnp.matmul(C, w)
    return w.reshape(_R, in_f, out_f)


def kernel(all_class_names, node_states, all_edge_ids, all_edge_types,
           mask_nodes, mask_edges, feat_in, W0, C0, W1, C1, W2, C2):
    ids = all_class_names[0].astype(jnp.int32)
    src = all_edge_ids[0, :, 0].astype(jnp.int32)
    dst = all_edge_ids[0, :, 1].astype(jnp.int32)
    rel = all_edge_types[0].astype(jnp.int32)

    ids_p = jnp.pad(ids, (0, _NP - _N)).reshape(_NW, _INCH, _ICH)
    epw0 = _E // _NW
    gidx = (rel * _NP + src).reshape(_NW, epw0)
    gidx = jnp.pad(gidx, ((0, 0), (0, _EPW - epw0))).reshape(_NW, _NCH, _ECH)
    didx = jnp.pad(dst.reshape(_NW, epw0), ((0, 0), (0, _EPW - epw0)),
                   constant_values=_N).reshape(_NW, _NCH, _ECH)

    z128 = jnp.zeros((_NP, _H), jnp.float32)
    w0 = _basis(C0, W0, _H, _H)
    w1 = _basis(C1, W1, _H, _H)
    w2 = jnp.pad(_basis(C2, W2, _H, _OUT), ((0, 0), (0, 0), (0, _H - _OUT)))

    h0 = _emb(feat_in, ids_p)
    hw0 = _tfirst(h0, w0).reshape(_R * _NP, _H)
    p0 = _gsa128(hw0, gidx, didx, z128)
    hw1 = _tmid(p0, w1).reshape(_R * _NP, _H)
    p1 = _gsa128(hw1, gidx, didx, z128)
    hw2 = _tmid(p1, w2).reshape(_R * _NP, _H)
    p2 = _gsa128(hw2, gidx, didx, z128)
    out = _softmax(p2)
    return out[:_N][None]
